# Initial kernel scaffold; baseline (speedup 1.0000x reference)
import jax
import jax.numpy as jnp
from jax.experimental import pallas as pl


def kernel(tokens, tok_emb, pos_emb):
    raise NotImplementedError("write your pallas kernel here")


# speedup vs baseline: 1.2376x; 1.2376x over previous
import functools

import jax
import jax.numpy as jnp
from jax import lax
from jax.experimental import pallas as pl
from jax.experimental.pallas import tpu as pltpu
from jax.experimental.pallas import tpu_sc as plsc

B, S, E = 4, 8192, 128
NW = 32
ROWS_PER_W = (B * S) // NW
CHUNK = 128
NCH = ROWS_PER_W // CHUNK
S_BLOCKS = S // ROWS_PER_W


def _emb_body(tok_hbm, tok_emb_hbm, pos_emb_hbm, out_hbm, idx_v, bufs, sems):
    c = lax.axis_index("c")
    s = lax.axis_index("s")
    wid = s * 2 + c
    base = wid * ROWS_PER_W
    s0 = (wid % S_BLOCKS) * ROWS_PER_W

    pltpu.sync_copy(tok_hbm.at[wid], idx_v)

    nbuf = len(bufs)
    for j in range(nbuf):
        pltpu.async_copy(
            pos_emb_hbm.at[pl.ds(s0 + j * CHUNK, CHUNK)], bufs[j], sems[j]
        )

    for j in range(NCH):
        b = j % nbuf
        pltpu.make_async_copy(
            pos_emb_hbm.at[pl.ds(s0 + j * CHUNK, CHUNK)], bufs[b], sems[b]
        ).wait()
        pltpu.async_copy(
            tok_emb_hbm.at[idx_v.at[j]], bufs[b], sems[b], add=True
        ).wait()
        pltpu.sync_copy(bufs[b], out_hbm.at[pl.ds(base + j * CHUNK, CHUNK)])
        nxt = j + nbuf
        if nxt < NCH:
            pltpu.async_copy(
                pos_emb_hbm.at[pl.ds(s0 + nxt * CHUNK, CHUNK)], bufs[b], sems[b]
            )


@functools.partial(jax.jit, static_argnames=())
def _emb(tokens_flat, tok_emb, pos_emb):
    nbuf = 2
    mesh = plsc.VectorSubcoreMesh(core_axis_name="c", subcore_axis_name="s")
    run = pl.kernel(
        _emb_body,
        out_type=jax.ShapeDtypeStruct((B * S, E), jnp.float32),
        mesh=mesh,
        scratch_types=[
            pltpu.VMEM((NCH, CHUNK), jnp.int32),
            [pltpu.VMEM((CHUNK, E), jnp.float32) for _ in range(nbuf)],
            [pltpu.SemaphoreType.DMA for _ in range(nbuf)],
        ],
    )
    return run(tokens_flat, tok_emb, pos_emb)


def kernel(tokens, tok_emb, pos_emb):
    tokens_flat = tokens.astype(jnp.int32).reshape(NW, NCH, CHUNK)
    out = _emb(tokens_flat, tok_emb, pos_emb)
    return out.reshape(B, S, E)


# speedup vs baseline: 1.3638x; 1.1020x over previous
import functools

import jax
import jax.numpy as jnp
from jax import lax
from jax.experimental import pallas as pl
from jax.experimental.pallas import tpu as pltpu
from jax.experimental.pallas import tpu_sc as plsc

B, S, E = 4, 8192, 128
NW = 32
ROWS_PER_W = (B * S) // NW
CHUNK = 128
NCH = ROWS_PER_W // CHUNK
S_BLOCKS = S // ROWS_PER_W
NBUF = 7


def _emb_body(tok_hbm, tok_emb_hbm, pos_emb_hbm, out_hbm,
              idx_v, bufs, psems, gsems, ssems):
    c = lax.axis_index("c")
    s = lax.axis_index("s")
    wid = s * 2 + c
    base = wid * ROWS_PER_W
    s0 = (wid % S_BLOCKS) * ROWS_PER_W

    pltpu.sync_copy(tok_hbm.at[wid], idx_v)

    def start_pos(j):
        return pltpu.async_copy(
            pos_emb_hbm.at[pl.ds(s0 + j * CHUNK, CHUNK)],
            bufs[j % NBUF], psems[j % NBUF])

    def start_gather(j):
        return pltpu.async_copy(
            tok_emb_hbm.at[idx_v.at[j]], bufs[j % NBUF], gsems[j % NBUF],
            add=True)

    def start_store(j):
        return pltpu.async_copy(
            bufs[j % NBUF], out_hbm.at[pl.ds(base + j * CHUNK, CHUNK)],
            ssems[j % NBUF])

    pos_d = [None] * NCH
    gat_d = [None] * NCH
    st_d = [None] * NCH

    for j in range(NBUF):
        pos_d[j] = start_pos(j)

    for j in range(NCH):
        pos_d[j].wait()
        gat_d[j] = start_gather(j)
        if j >= 1:
            gat_d[j - 1].wait()
            st_d[j - 1] = start_store(j - 1)
        if j >= 2 and (j - 2) + NBUF < NCH:
            st_d[j - 2].wait()
            pos_d[(j - 2) + NBUF] = start_pos((j - 2) + NBUF)

    gat_d[NCH - 1].wait()
    st_d[NCH - 1] = start_store(NCH - 1)
    for j in range(NCH):
        if j + NBUF >= NCH:
            st_d[j].wait()


@jax.jit
def _emb(tokens_flat, tok_emb, pos_emb):
    mesh = plsc.VectorSubcoreMesh(core_axis_name="c", subcore_axis_name="s")
    run = pl.kernel(
        _emb_body,
        out_type=jax.ShapeDtypeStruct((B * S, E), jnp.float32),
        mesh=mesh,
        scratch_types=[
            pltpu.VMEM((NCH, CHUNK), jnp.int32),
            [pltpu.VMEM((CHUNK, E), jnp.float32) for _ in range(NBUF)],
            [pltpu.SemaphoreType.DMA for _ in range(NBUF)],
            [pltpu.SemaphoreType.DMA for _ in range(NBUF)],
            [pltpu.SemaphoreType.DMA for _ in range(NBUF)],
        ],
    )
    return run(tokens_flat, tok_emb, pos_emb)


def kernel(tokens, tok_emb, pos_emb):
    tokens_flat = tokens.astype(jnp.int32).reshape(NW, NCH, CHUNK)
    out = _emb(tokens_flat, tok_emb, pos_emb)
    return out.reshape(B, S, E)


# speedup vs baseline: 1.3928x; 1.0212x over previous
import jax
import jax.numpy as jnp
from jax import lax
from jax.experimental import pallas as pl
from jax.experimental.pallas import tpu as pltpu
from jax.experimental.pallas import tpu_sc as plsc

B, S, E = 4, 8192, 128
NW = 32
SW = S // NW
CHUNK = 128
HB = SW // CHUNK
NCH = B * HB
NBUF = 5
LANES = 16


def _emb_body(tok_hbm, tok_emb_hbm, pos_emb_hbm, out_hbm,
              idx_v, pos_v, bufs, psem, gsems, ssems):
    c = lax.axis_index("c")
    s = lax.axis_index("s")
    wid = s * 2 + c
    s0 = wid * SW

    pos_d = pltpu.async_copy(pos_emb_hbm.at[pl.ds(s0, SW)], pos_v, psem)

    for b in range(B):
        pltpu.sync_copy(tok_hbm.at[b, pl.ds(wid * HB, HB)],
                        idx_v.at[pl.ds(b * HB, HB)])

    def start_gather(j):
        return pltpu.async_copy(
            tok_emb_hbm.at[idx_v.at[j]], bufs[j % NBUF], gsems[j % NBUF])

    def start_store(j):
        b, h = divmod(j, HB)
        off = b * S + s0 + h * CHUNK
        return pltpu.async_copy(
            bufs[j % NBUF], out_hbm.at[pl.ds(off, CHUNK)], ssems[j % NBUF])

    gat_d = [None] * NCH
    st_d = [None] * NCH

    for j in range(NBUF):
        gat_d[j] = start_gather(j)

    pos_d.wait()
    for j in range(NCH):
        gat_d[j].wait()
        buf = bufs[j % NBUF]
        h0 = (j % HB) * CHUNK

        @plsc.parallel_loop(0, CHUNK, step=1, unroll=4)
        def add_pos(r):
            for k in range(E // LANES):
                sl = pl.ds(k * LANES, LANES)
                buf[r, sl] = buf[r, sl] + pos_v[h0 + r, sl]

        st_d[j] = start_store(j)
        if j >= 1 and (j - 1) + NBUF < NCH:
            st_d[j - 1].wait()
            gat_d[(j - 1) + NBUF] = start_gather((j - 1) + NBUF)

    for j in range(NCH):
        if j + NBUF >= NCH:
            st_d[j].wait()


@jax.jit
def _emb(tokens, tok_emb, pos_emb):
    mesh = plsc.VectorSubcoreMesh(core_axis_name="c", subcore_axis_name="s")
    run = pl.kernel(
        _emb_body,
        out_type=jax.ShapeDtypeStruct((B * S, E), jnp.float32),
        mesh=mesh,
        scratch_types=[
            pltpu.VMEM((NCH, CHUNK), jnp.int32),
            pltpu.VMEM((SW, E), jnp.float32),
            [pltpu.VMEM((CHUNK, E), jnp.float32) for _ in range(NBUF)],
            pltpu.SemaphoreType.DMA,
            [pltpu.SemaphoreType.DMA for _ in range(NBUF)],
            [pltpu.SemaphoreType.DMA for _ in range(NBUF)],
        ],
    )
    return run(tokens, tok_emb, pos_emb)


def kernel(tokens, tok_emb, pos_emb):
    tokens_r = tokens.astype(jnp.int32).reshape(B, S // CHUNK, CHUNK)
    out = _emb(tokens_r, tok_emb, pos_emb)
    return out.reshape(B, S, E)
